# Initial kernel scaffold; baseline (speedup 1.0000x reference)
import jax
import jax.numpy as jnp
from jax.experimental import pallas as pl


def kernel(x, table):
    raise NotImplementedError("write your pallas kernel here")


# speedup vs baseline: 4.5698x; 4.5698x over previous
import math
import functools

import jax
import jax.numpy as jnp
from jax import lax
from jax.experimental import pallas as pl
from jax.experimental.pallas import tpu as pltpu
from jax.experimental.pallas import tpu_sc as plsc

VOCAB_D = 32
SCALE = math.sqrt(VOCAB_D)

NC = 2
NS = 16
NW = NC * NS

CHUNK = 1024


def _body(x_hbm, tab_hbm, out_hbm, idx_v, rows_v, sem):
    wid = lax.axis_index("s") * NC + lax.axis_index("c")
    n_per_w = x_hbm.shape[0] // NW
    n_chunks = n_per_w // CHUNK
    base = wid * n_per_w

    def step(g, carry):
        off = base + g * CHUNK
        pltpu.sync_copy(x_hbm.at[pl.ds(off, CHUNK)], idx_v)
        pltpu.async_copy(tab_hbm.at[idx_v], rows_v, sem).wait()

        def scale_body(i, c):
            r0 = i * 8
            for j in range(8):
                for k in range(2):
                    sl = (r0 + j, pl.ds(k * 16, 16))
                    rows_v[sl] = rows_v[sl] * SCALE
            return c

        lax.fori_loop(0, CHUNK // 8, scale_body, 0)
        pltpu.sync_copy(rows_v, out_hbm.at[pl.ds(off, CHUNK)])
        return carry

    lax.fori_loop(0, n_chunks, step, 0)


def kernel(x, table):
    n = x.shape[0] * x.shape[1]
    xf = x.reshape(n).astype(jnp.int32)
    mesh = plsc.VectorSubcoreMesh(core_axis_name="c", subcore_axis_name="s")
    run = pl.kernel(
        _body,
        out_type=jax.ShapeDtypeStruct((n, VOCAB_D), jnp.float32),
        mesh=mesh,
        scratch_types=[
            pltpu.VMEM((CHUNK,), jnp.int32),
            pltpu.VMEM((CHUNK, VOCAB_D), jnp.float32),
            pltpu.SemaphoreType.DMA,
        ],
        compiler_params=pltpu.CompilerParams(use_tc_tiling_on_sc=False),
    )
    out = run(xf, table)
    return out.reshape(x.shape[0], x.shape[1], VOCAB_D)


# speedup vs baseline: 5.0119x; 1.0967x over previous
import math

import jax
import jax.numpy as jnp
from jax import lax
from jax.experimental import pallas as pl
from jax.experimental.pallas import tpu as pltpu
from jax.experimental.pallas import tpu_sc as plsc

D_EMB = 32
SCALE = math.sqrt(D_EMB)

NC = 2
NS = 16
NW = NC * NS

CHUNK = 1024
ROWS_PER_ITER = 16


def _body(x_hbm, tab_hbm, out_hbm,
          idx0, idx1, rows0, rows1,
          isem0, isem1, gsem0, gsem1, ssem0, ssem1):
    idx = (idx0, idx1)
    rows = (rows0, rows1)
    isem = (isem0, isem1)
    gsem = (gsem0, gsem1)
    ssem = (ssem0, ssem1)

    wid = lax.axis_index("s") * NC + lax.axis_index("c")
    n_per_w = x_hbm.shape[0] // NW
    n_chunks = n_per_w // CHUNK
    base = wid * n_per_w

    def x_slice(g):
        return x_hbm.at[pl.ds(base + g * CHUNK, CHUNK)]

    def out_slice(g):
        return out_hbm.at[pl.ds(base + g * CHUNK, CHUNK)]

    def scale(b):
        def scale_body(i, c):
            r0 = i * ROWS_PER_ITER
            for j in range(ROWS_PER_ITER):
                for k in range(2):
                    sl = (r0 + j, pl.ds(k * 16, 16))
                    rows[b][sl] = rows[b][sl] * SCALE
            return c

        lax.fori_loop(0, CHUNK // ROWS_PER_ITER, scale_body, 0)

    pltpu.async_copy(x_slice(0), idx[0], isem[0])
    pltpu.async_copy(x_slice(1), idx[1], isem[1])
    pltpu.make_async_copy(x_slice(0), idx[0], isem[0]).wait()
    pltpu.async_copy(tab_hbm.at[idx[0]], rows[0], gsem[0])

    def outer(t, carry):
        for b in range(2):
            g = 2 * t + b
            nb = 1 - b

            @pl.when(g + 1 < n_chunks)
            def _():
                pltpu.make_async_copy(x_slice(g + 1), idx[nb], isem[nb]).wait()

                @pl.when(g >= 1)
                def _():
                    pltpu.make_async_copy(
                        rows[nb], out_slice(g - 1), ssem[nb]).wait()

                pltpu.async_copy(tab_hbm.at[idx[nb]], rows[nb], gsem[nb])

            pltpu.make_async_copy(tab_hbm.at[idx[b]], rows[b], gsem[b]).wait()

            @pl.when(g + 2 < n_chunks)
            def _():
                pltpu.async_copy(x_slice(g + 2), idx[b], isem[b])

            scale(b)
            pltpu.async_copy(rows[b], out_slice(g), ssem[b])
        return carry

    lax.fori_loop(0, n_chunks // 2, outer, 0)

    pltpu.make_async_copy(rows[0], out_slice(n_chunks - 2), ssem[0]).wait()
    pltpu.make_async_copy(rows[1], out_slice(n_chunks - 1), ssem[1]).wait()


def kernel(x, table):
    n = x.shape[0] * x.shape[1]
    xf = x.reshape(n).astype(jnp.int32)
    mesh = plsc.VectorSubcoreMesh(core_axis_name="c", subcore_axis_name="s")
    run = pl.kernel(
        _body,
        out_type=jax.ShapeDtypeStruct((n, D_EMB), jnp.float32),
        mesh=mesh,
        scratch_types=[
            pltpu.VMEM((CHUNK,), jnp.int32),
            pltpu.VMEM((CHUNK,), jnp.int32),
            pltpu.VMEM((CHUNK, D_EMB), jnp.float32),
            pltpu.VMEM((CHUNK, D_EMB), jnp.float32),
            pltpu.SemaphoreType.DMA,
            pltpu.SemaphoreType.DMA,
            pltpu.SemaphoreType.DMA,
            pltpu.SemaphoreType.DMA,
            pltpu.SemaphoreType.DMA,
            pltpu.SemaphoreType.DMA,
        ],
        compiler_params=pltpu.CompilerParams(use_tc_tiling_on_sc=False),
    )
    out = run(xf, table)
    return out.reshape(x.shape[0], x.shape[1], D_EMB)


# speedup vs baseline: 7.5827x; 1.5129x over previous
import math

import jax
import jax.numpy as jnp
from jax import lax
from jax.experimental import pallas as pl
from jax.experimental.pallas import tpu as pltpu
from jax.experimental.pallas import tpu_sc as plsc

D_EMB = 32
SCALE = math.sqrt(D_EMB)

NC = 2
NS = 16
NW = NC * NS

CHUNK = 512
PITCH = CHUNK + 1
R_UNROLL = 8


def _body(xf_hbm, tab_hbm, out_hbm,
          idx0, idx1, rows0, rows1, rt0, rt1,
          isem0, isem1, gsem0, gsem1, ssem0, ssem1):
    idx = (idx0, idx1)
    rows = (rows0, rows1)
    rt = (rt0, rt1)
    isem = (isem0, isem1)
    gsem = (gsem0, gsem1)
    ssem = (ssem0, ssem1)

    n = xf_hbm.shape[0]
    a_len = out_hbm.shape[2]
    chunks_per_j = a_len // CHUNK
    n_chunks = (n // CHUNK) // NW
    wid = lax.axis_index("s") * NC + lax.axis_index("c")
    c_base = wid * n_chunks

    iota = lax.iota(jnp.int32, 16)
    iota_hi = iota + 16

    def x_slice(g):
        return xf_hbm.at[pl.ds((c_base + g) * CHUNK, CHUNK)]

    def out_slice(g):
        c = c_base + g
        j = c // chunks_per_j
        a0 = (c % chunks_per_j) * CHUNK
        return out_hbm.at[j, :, pl.ds(a0, CHUNK)]

    def rt_src(b):
        return rt[b].at[:, pl.ds(0, CHUNK)]

    def transpose_scale(b):
        def tr_body(i, carry):
            r0 = i * R_UNROLL
            for u in range(R_UNROLL):
                r = r0 + u
                rsp = jnp.full((16,), 0, jnp.int32) + r
                v0 = rows[b][r, pl.ds(0, 16)] * SCALE
                v1 = rows[b][r, pl.ds(16, 16)] * SCALE
                plsc.store_scatter(rt[b], [iota, rsp], v0)
                plsc.store_scatter(rt[b], [iota_hi, rsp], v1)
            return carry

        lax.fori_loop(0, CHUNK // R_UNROLL, tr_body, 0)

    pltpu.async_copy(x_slice(0), idx[0], isem[0])
    pltpu.async_copy(x_slice(1), idx[1], isem[1])
    pltpu.make_async_copy(x_slice(0), idx[0], isem[0]).wait()
    pltpu.async_copy(tab_hbm.at[idx[0]], rows[0], gsem[0])

    def outer(t, carry):
        for b in range(2):
            g = 2 * t + b
            nb = 1 - b

            @pl.when(g + 1 < n_chunks)
            def _():
                pltpu.make_async_copy(x_slice(g + 1), idx[nb], isem[nb]).wait()

                @pl.when(g >= 1)
                def _():
                    pltpu.make_async_copy(
                        rt_src(nb), out_slice(g - 1), ssem[nb]).wait()

                pltpu.async_copy(tab_hbm.at[idx[nb]], rows[nb], gsem[nb])

            pltpu.make_async_copy(tab_hbm.at[idx[b]], rows[b], gsem[b]).wait()

            @pl.when(g + 2 < n_chunks)
            def _():
                pltpu.async_copy(x_slice(g + 2), idx[b], isem[b])

            transpose_scale(b)
            pltpu.async_copy(rt_src(b), out_slice(g), ssem[b])
        return carry

    lax.fori_loop(0, n_chunks // 2, outer, 0)

    pltpu.make_async_copy(rt_src(0), out_slice(n_chunks - 2), ssem[0]).wait()
    pltpu.make_async_copy(rt_src(1), out_slice(n_chunks - 1), ssem[1]).wait()


def kernel(x, table):
    n_tok, n_seq = x.shape
    n = n_tok * n_seq
    xf = x.T.reshape(n).astype(jnp.int32)
    mesh = plsc.VectorSubcoreMesh(core_axis_name="c", subcore_axis_name="s")
    run = pl.kernel(
        _body,
        out_type=jax.ShapeDtypeStruct((n_seq, D_EMB, n_tok), jnp.float32),
        mesh=mesh,
        scratch_types=[
            pltpu.VMEM((CHUNK,), jnp.int32),
            pltpu.VMEM((CHUNK,), jnp.int32),
            pltpu.VMEM((CHUNK, D_EMB), jnp.float32),
            pltpu.VMEM((CHUNK, D_EMB), jnp.float32),
            pltpu.VMEM((D_EMB, PITCH), jnp.float32),
            pltpu.VMEM((D_EMB, PITCH), jnp.float32),
            pltpu.SemaphoreType.DMA,
            pltpu.SemaphoreType.DMA,
            pltpu.SemaphoreType.DMA,
            pltpu.SemaphoreType.DMA,
            pltpu.SemaphoreType.DMA,
            pltpu.SemaphoreType.DMA,
        ],
        compiler_params=pltpu.CompilerParams(
            use_tc_tiling_on_sc=False, needs_layout_passes=False),
    )
    out_p = run(xf, table)
    return out_p.transpose(2, 0, 1)
